# baseline (speedup 1.0000x reference)
import jax
import jax.numpy as jnp
from jax.experimental import pallas as pl
from jax.experimental.pallas import tpu as pltpu

B = 16
NQ = 20000
NCLS = 80
FLAT = NQ * NCLS
NROW = 12500
NLANE = 128
GS = 16
NGRP = FLAT // GS
KGRP = 512
NCAND = KGRP * GS
KOUT = 300
BOXROW = NQ * 4 // NLANE


def _f32(x):
    return x.astype(jnp.float32)


def _amax_body(x_ref, o_ref):
    x = x_ref[0]
    w = jnp.maximum(x, jnp.concatenate([x[:, 8:], x[:, :8]], axis=1))
    w = jnp.maximum(w, jnp.concatenate([w[:, 4:], w[:, :4]], axis=1))
    w = jnp.maximum(w, jnp.concatenate([w[:, 2:], w[:, :2]], axis=1))
    w = jnp.maximum(w, jnp.concatenate([w[:, 1:], w[:, :1]], axis=1))
    li = jax.lax.broadcasted_iota(jnp.int32, (NLANE, NLANE // GS), 0)
    ci = jax.lax.broadcasted_iota(jnp.int32, (NLANE, NLANE // GS), 1)
    sel = _f32(li == GS * ci)
    o_ref[0] = jax.lax.dot_general(
        w, sel, (((1,), (0,)), ((), ())), preferred_element_type=jnp.float32)


def _group_max(x_rows):
    return pl.pallas_call(
        _amax_body,
        grid=(B,),
        in_specs=[pl.BlockSpec((1, NROW, NLANE), lambda b: (b, 0, 0))],
        out_specs=pl.BlockSpec((1, NROW, NLANE // GS), lambda b: (b, 0, 0)),
        out_shape=jax.ShapeDtypeStruct((B, NROW, NLANE // GS), jnp.float32),
    )(x_rows)


def _gather_body(gsm, x_ref, gv_ref, bx_ref, oc_ref, ob_ref, rbuf, bbuf):
    b = pl.program_id(0)

    def body(i, _):
        g = gsm[b * KGRP + i]
        rbuf[pl.ds(i, 1), :] = x_ref[0, pl.ds(g // (NLANE // GS), 1), :]
        q = g // (NCLS // GS)
        bbuf[pl.ds(i, 1), :] = bx_ref[0, pl.ds(q // 32, 1), :]
        return 0

    jax.lax.fori_loop(0, KGRP, body, 0)
    gcol = gv_ref[0]
    rows = rbuf[...]
    pcol = gcol % (NLANE // GS)
    acc = rows[:, 0:GS]
    for p in range(1, NLANE // GS):
        acc = jnp.where(pcol == p, rows[:, GS * p:GS * p + GS], acc)
    oc_ref[0] = acc
    brows = bbuf[...]
    qcol = (gcol // (NCLS // GS)) % 32
    bacc = brows[:, 0:4]
    for p in range(1, 32):
        bacc = jnp.where(qcol == p, brows[:, 4 * p:4 * p + 4], bacc)
    ob_ref[0] = bacc


def _gather_candidates(x_rows, box_rows, gids):
    grid_spec = pltpu.PrefetchScalarGridSpec(
        num_scalar_prefetch=1,
        grid=(B,),
        in_specs=[
            pl.BlockSpec((1, NROW, NLANE), lambda b, g: (b, 0, 0)),
            pl.BlockSpec((1, KGRP, 1), lambda b, g: (b, 0, 0)),
            pl.BlockSpec((1, BOXROW, NLANE), lambda b, g: (b, 0, 0)),
        ],
        out_specs=[
            pl.BlockSpec((1, KGRP, GS), lambda b, g: (b, 0, 0)),
            pl.BlockSpec((1, KGRP, 4), lambda b, g: (b, 0, 0)),
        ],
        scratch_shapes=[
            pltpu.VMEM((KGRP, NLANE), jnp.float32),
            pltpu.VMEM((KGRP, NLANE), jnp.float32),
        ],
    )
    return pl.pallas_call(
        _gather_body,
        grid_spec=grid_spec,
        out_shape=[
            jax.ShapeDtypeStruct((B, KGRP, GS), jnp.float32),
            jax.ShapeDtypeStruct((B, KGRP, 4), jnp.float32),
        ],
    )(gids.reshape(-1), x_rows, gids.reshape(B, KGRP, 1), box_rows)


def kernel(pred_logits, pred_boxes, orig_target_sizes):
    x_rows = pred_logits.reshape(B, NROW, NLANE)
    box_rows = pred_boxes.reshape(B, BOXROW, NLANE)
    m = _group_max(x_rows).reshape(B, NGRP)
    _, gids = jax.lax.top_k(m, KGRP)
    gids = jnp.sort(gids, axis=1)
    cand_logits, cand_boxes = _gather_candidates(x_rows, box_rows, gids)
    scores = jax.nn.sigmoid(cand_logits).reshape(B, NCAND)
    flat_idx = (gids[:, :, None] * GS
                + jnp.arange(GS, dtype=jnp.int32)).reshape(B, NCAND)
    topk_scores, pos = jax.lax.top_k(scores, KOUT)
    fi = jnp.take_along_axis(flat_idx, pos, axis=1)
    labels = fi % NCLS
    raw = jnp.take_along_axis(
        cand_boxes, jnp.broadcast_to((pos // GS)[:, :, None], (B, KOUT, 4)),
        axis=1)
    cx, cy, w, h = raw[..., 0], raw[..., 1], raw[..., 2], raw[..., 3]
    bbox = jnp.stack(
        [cx - 0.5 * w, cy - 0.5 * h, cx + 0.5 * w, cy + 0.5 * h], axis=-1)
    scale = jnp.tile(orig_target_sizes.astype(jnp.float32), (1, 2))[:, None, :]
    return (labels, bbox * scale, topk_scores)
